# baseline (speedup 1.0000x reference)
import jax
import jax.numpy as jnp
from jax import lax
from jax.experimental import pallas as pl
from jax.experimental.pallas import tpu as pltpu
from jax.experimental.pallas import tpu_sc as plsc

VOCAB = 100000
D = 1024
B = 4
SEQ = 2048
N_TOK = B * SEQ

NC = 2
NS = 16
NW = NC * NS
LANES = 16

POS_PER_W = SEQ // NW
CS = 8
NSS = POS_PER_W // CS
RING = 3
PER = 2
KU = 16


def _body(ids_hbm, table_hbm, pe_hbm, out_hbm,
          pe_a, pe_b, idx_all, sb0, sb1, sb2,
          g0, g1, g2, st0, st1, st2, psem):
    c = lax.axis_index("c")
    s = lax.axis_index("s")
    wid = s * NC + c
    wpos = wid * POS_PER_W

    sbuf = [sb0, sb1, sb2]
    gsem = [g0, g1, g2]
    ssem = [st0, st1, st2]
    pebuf = [pe_a, pe_b]

    def pe_fetch(sp):
        return pltpu.async_copy(
            pe_hbm.at[pl.ds(wpos + sp * CS, CS)],
            pebuf[sp % PER], psem)

    def fire_gathers(sp):
        ring = sp % RING
        sb = sbuf[ring]
        return [pltpu.async_copy(
                    table_hbm.at[idx_all.at[b, pl.ds(sp * CS, CS)]],
                    sb.at[pl.ds(b * CS, CS)], gsem[ring])
                for b in range(B)]

    pe_cps = {0: pe_fetch(0), 1: pe_fetch(1)}
    for b in range(B):
        pltpu.sync_copy(ids_hbm.at[pl.ds(b * SEQ + wpos, POS_PER_W)],
                        idx_all.at[b])

    gthr = {sp: fire_gathers(sp) for sp in range(RING)}
    stores = {}

    for sp in range(NSS):
        ring = sp % RING
        sb = sbuf[ring]
        pe_v = pebuf[sp % PER]
        for cp in gthr[sp]:
            cp.wait()
        pe_cps[sp].wait()

        def kblock(kb, carry, sb=sb, pe_v=pe_v):
            @plsc.parallel_loop(0, CS, 1)
            def add_body(r):
                for kk in range(KU):
                    sl = pl.ds(kb * (KU * LANES) + kk * LANES, LANES)
                    v = pe_v[r, sl]
                    for b in range(B):
                        sb[b * CS + r, sl] = sb[b * CS + r, sl] + v
            return carry

        lax.fori_loop(0, D // (KU * LANES), kblock, 0)

        if sp + PER < NSS:
            pe_cps[sp + PER] = pe_fetch(sp + PER)

        stores[sp] = [pltpu.async_copy(
                          sb.at[pl.ds(b * CS, CS)],
                          out_hbm.at[pl.ds(b * SEQ + wpos + sp * CS, CS)],
                          ssem[ring])
                      for b in range(B)]

        if sp + RING < NSS:
            for cp in stores[sp]:
                cp.wait()
            gthr[sp + RING] = fire_gathers(sp + RING)

    for sp in range(NSS - RING, NSS):
        for cp in stores[sp]:
            cp.wait()


def kernel(input_ids, table, pe):
    ids_flat = input_ids.reshape(N_TOK).astype(jnp.int32)
    mesh = plsc.VectorSubcoreMesh(core_axis_name="c", subcore_axis_name="s")
    out = pl.kernel(
        _body,
        mesh=mesh,
        out_type=jax.ShapeDtypeStruct((N_TOK, D), jnp.float32),
        scratch_types=[
            pltpu.VMEM((CS, D), jnp.float32),
            pltpu.VMEM((CS, D), jnp.float32),
            pltpu.VMEM((B, POS_PER_W), jnp.int32),
            pltpu.VMEM((B * CS, D), jnp.float32),
            pltpu.VMEM((B * CS, D), jnp.float32),
            pltpu.VMEM((B * CS, D), jnp.float32),
            pltpu.SemaphoreType.DMA,
            pltpu.SemaphoreType.DMA,
            pltpu.SemaphoreType.DMA,
            pltpu.SemaphoreType.DMA,
            pltpu.SemaphoreType.DMA,
            pltpu.SemaphoreType.DMA,
            pltpu.SemaphoreType.DMA,
        ],
    )(ids_flat, table, pe)
    return out.reshape(B, SEQ, D)


# speedup vs baseline: 1.2092x; 1.2092x over previous
import jax
import jax.numpy as jnp
from jax import lax
from jax.experimental import pallas as pl
from jax.experimental.pallas import tpu as pltpu
from jax.experimental.pallas import tpu_sc as plsc

VOCAB = 100000
D = 1024
B = 4
SEQ = 2048
N_TOK = B * SEQ

NC = 2
NS = 16
NW = NC * NS
LANES = 16

POS_PER_W = SEQ // NW
CS = 8
NSS = POS_PER_W // CS
RING = 3
PER = 2


def _body(ids_hbm, table_hbm, pe_hbm, out_hbm,
          pe_a, pe_b, idx_all, sb0, sb1, sb2,
          g0, g1, g2, st0, st1, st2, psem):
    c = lax.axis_index("c")
    s = lax.axis_index("s")
    wid = s * NC + c
    wpos = wid * POS_PER_W

    sbuf = [sb0, sb1, sb2]
    gsem = [g0, g1, g2]
    ssem = [st0, st1, st2]
    pebuf = [pe_a, pe_b]

    def pe_fetch(sp):
        return pltpu.async_copy(
            pe_hbm.at[pl.ds(wpos + sp * CS, CS)],
            pebuf[sp % PER], psem)

    def fire_gathers(sp):
        ring = sp % RING
        sb = sbuf[ring]
        return [pltpu.async_copy(
                    table_hbm.at[idx_all.at[b, pl.ds(sp * CS, CS)]],
                    sb.at[pl.ds(b * CS, CS)], gsem[ring])
                for b in range(B)]

    pe_cps = {0: pe_fetch(0), 1: pe_fetch(1)}
    for b in range(B):
        pltpu.sync_copy(ids_hbm.at[pl.ds(b * SEQ + wpos, POS_PER_W)],
                        idx_all.at[b])

    gthr = {sp: fire_gathers(sp) for sp in range(RING)}
    stores = {}

    for sp in range(NSS):
        ring = sp % RING
        sb = sbuf[ring]
        pe_v = pebuf[sp % PER]
        for cp in gthr[sp]:
            cp.wait()
        pe_cps[sp].wait()

        @plsc.parallel_loop(0, D // LANES, 1)
        def add_body(kq, sb=sb, pe_v=pe_v):
            sl = pl.ds(kq * LANES, LANES)
            for r in range(CS):
                v = pe_v[r, sl]
                for b in range(B):
                    sb[b * CS + r, sl] = sb[b * CS + r, sl] + v

        if sp + PER < NSS:
            pe_cps[sp + PER] = pe_fetch(sp + PER)

        stores[sp] = [pltpu.async_copy(
                          sb.at[pl.ds(b * CS, CS)],
                          out_hbm.at[pl.ds(b * SEQ + wpos + sp * CS, CS)],
                          ssem[ring])
                      for b in range(B)]

        if sp + RING < NSS:
            for cp in stores[sp]:
                cp.wait()
            gthr[sp + RING] = fire_gathers(sp + RING)

    for sp in range(NSS - RING, NSS):
        for cp in stores[sp]:
            cp.wait()


def kernel(input_ids, table, pe):
    ids_flat = input_ids.reshape(N_TOK).astype(jnp.int32)
    mesh = plsc.VectorSubcoreMesh(core_axis_name="c", subcore_axis_name="s")
    out = pl.kernel(
        _body,
        mesh=mesh,
        out_type=jax.ShapeDtypeStruct((N_TOK, D), jnp.float32),
        scratch_types=[
            pltpu.VMEM((CS, D), jnp.float32),
            pltpu.VMEM((CS, D), jnp.float32),
            pltpu.VMEM((B, POS_PER_W), jnp.int32),
            pltpu.VMEM((B * CS, D), jnp.float32),
            pltpu.VMEM((B * CS, D), jnp.float32),
            pltpu.VMEM((B * CS, D), jnp.float32),
            pltpu.SemaphoreType.DMA,
            pltpu.SemaphoreType.DMA,
            pltpu.SemaphoreType.DMA,
            pltpu.SemaphoreType.DMA,
            pltpu.SemaphoreType.DMA,
            pltpu.SemaphoreType.DMA,
            pltpu.SemaphoreType.DMA,
        ],
    )(ids_flat, table, pe)
    return out.reshape(B, SEQ, D)


# speedup vs baseline: 1.2414x; 1.0266x over previous
import jax
import jax.numpy as jnp
from jax import lax
from jax.experimental import pallas as pl
from jax.experimental.pallas import tpu as pltpu
from jax.experimental.pallas import tpu_sc as plsc

VOCAB = 100000
D = 1024
B = 4
SEQ = 2048
N_TOK = B * SEQ

NC = 2
NS = 16
NW = NC * NS
LANES = 16

POS_PER_W = SEQ // NW
CS = 8
NSS = POS_PER_W // CS
RING = 3
PER = 2


def _body(ids_hbm, table_hbm, pe_hbm, out_hbm,
          pe_a, pe_b, idx_all, sb0, sb1, sb2,
          g0, g1, g2, st0, st1, st2, psem, isem):
    c = lax.axis_index("c")
    s = lax.axis_index("s")
    wid = s * NC + c
    wpos = wid * POS_PER_W

    sbuf = [sb0, sb1, sb2]
    gsem = [g0, g1, g2]
    ssem = [st0, st1, st2]
    pebuf = [pe_a, pe_b]

    def pe_fetch(sp):
        return pltpu.async_copy(
            pe_hbm.at[pl.ds(wpos + sp * CS, CS)],
            pebuf[sp % PER], psem)

    def fire_gathers(sp):
        ring = sp % RING
        return [pltpu.async_copy(
                    table_hbm.at[idx_all.at[sp]],
                    sbuf[ring], gsem[ring])]

    pe_cps = {0: pe_fetch(0), 1: pe_fetch(1)}
    idx_cps = [pltpu.async_copy(
                   ids_hbm.at[pl.ds(b * SEQ + wpos + sp * CS, CS)],
                   idx_all.at[sp, pl.ds(b * CS, CS)], isem)
               for sp in range(NSS) for b in range(B)]
    for cp in idx_cps:
        cp.wait()

    gthr = {sp: fire_gathers(sp) for sp in range(RING)}
    stores = {}

    for sp in range(NSS):
        ring = sp % RING
        sb = sbuf[ring]
        pe_v = pebuf[sp % PER]
        for cp in gthr[sp]:
            cp.wait()
        pe_cps[sp].wait()

        @plsc.parallel_loop(0, D // LANES, 1)
        def add_body(kq, sb=sb, pe_v=pe_v):
            sl = pl.ds(kq * LANES, LANES)
            for r in range(CS):
                v = pe_v[r, sl]
                for b in range(B):
                    sb[b * CS + r, sl] = sb[b * CS + r, sl] + v

        if sp + PER < NSS:
            pe_cps[sp + PER] = pe_fetch(sp + PER)

        stores[sp] = [pltpu.async_copy(
                          sb.at[pl.ds(b * CS, CS)],
                          out_hbm.at[pl.ds(b * SEQ + wpos + sp * CS, CS)],
                          ssem[ring])
                      for b in range(B)]

        if sp + RING < NSS:
            for cp in stores[sp]:
                cp.wait()
            gthr[sp + RING] = fire_gathers(sp + RING)

    for sp in range(NSS - RING, NSS):
        for cp in stores[sp]:
            cp.wait()


def kernel(input_ids, table, pe):
    ids_flat = input_ids.reshape(N_TOK).astype(jnp.int32)
    mesh = plsc.VectorSubcoreMesh(core_axis_name="c", subcore_axis_name="s")
    out = pl.kernel(
        _body,
        mesh=mesh,
        out_type=jax.ShapeDtypeStruct((N_TOK, D), jnp.float32),
        scratch_types=[
            pltpu.VMEM((CS, D), jnp.float32),
            pltpu.VMEM((CS, D), jnp.float32),
            pltpu.VMEM((NSS, B * CS), jnp.int32),
            pltpu.VMEM((B * CS, D), jnp.float32),
            pltpu.VMEM((B * CS, D), jnp.float32),
            pltpu.VMEM((B * CS, D), jnp.float32),
            pltpu.SemaphoreType.DMA,
            pltpu.SemaphoreType.DMA,
            pltpu.SemaphoreType.DMA,
            pltpu.SemaphoreType.DMA,
            pltpu.SemaphoreType.DMA,
            pltpu.SemaphoreType.DMA,
            pltpu.SemaphoreType.DMA,
            pltpu.SemaphoreType.DMA,
        ],
    )(ids_flat, table, pe)
    return out.reshape(B, SEQ, D)
